# baseline (speedup 1.0000x reference)
import jax
import jax.numpy as jnp
from jax.experimental import pallas as pl

DIM = 256
NUM_Q = 2
CODEBOOK_SIZE = 256
COMMIT_W = 0.25
IN_DIM = 784


def _fused_body(x_ref, ew1_ref, eb1_ref, ew2_ref, eb2_ref, cb_ref,
                dw1_ref, db1_ref, dw2_ref, db2_ref,
                recon_ref, idx_ref, loss_ref, *, batch_total):
    i = pl.program_id(0)
    xb = x_ref[...]
    rows = xb.shape[0]

    h = jnp.dot(xb, ew1_ref[...], preferred_element_type=jnp.float32)
    h = jnp.maximum(h + eb1_ref[...], 0.0)
    e = jnp.dot(h, ew2_ref[...], preferred_element_type=jnp.float32)
    e = e + eb2_ref[...]

    lane = jax.lax.broadcasted_iota(jnp.int32, (rows, CODEBOOK_SIZE), 1)
    resid = e
    qsum = jnp.zeros_like(e)
    commit_sum = jnp.float32(0.0)
    idx_list = []
    for q in range(NUM_Q):
        cbq = cb_ref[q]
        cn = jnp.sum(cbq * cbq, axis=1)[None, :]
        rn = jnp.sum(resid * resid, axis=1, keepdims=True)
        rc = jax.lax.dot_general(resid, cbq, (((1,), (1,)), ((), ())),
                                 preferred_element_type=jnp.float32)
        d = rn - 2.0 * rc + cn
        dmin = jnp.min(d, axis=1, keepdims=True)
        idx = jnp.min(jnp.where(d == dmin, lane, CODEBOOK_SIZE), axis=1)
        onehot = (lane == idx[:, None]).astype(jnp.float32)
        quant = jax.lax.dot(onehot, cbq,
                            precision=jax.lax.Precision.HIGHEST)
        diff = resid - quant
        commit_sum = commit_sum + jnp.sum(diff * diff)
        qsum = qsum + quant
        resid = diff
        idx_list.append(idx)

    dh = jnp.dot(qsum, dw1_ref[...], preferred_element_type=jnp.float32)
    dh = jnp.maximum(dh + db1_ref[...], 0.0)
    rec = jnp.dot(dh, dw2_ref[...], preferred_element_type=jnp.float32)
    rec = jnp.tanh(rec + db2_ref[...])

    recon_ref[...] = rec
    idx_ref[...] = jnp.stack(idx_list, axis=1)

    dx = rec - xb
    part = (jnp.sum(dx * dx) / (batch_total * float(IN_DIM))
            + COMMIT_W * commit_sum / (batch_total * float(DIM)))

    @pl.when(i == 0)
    def _init():
        loss_ref[...] = jnp.zeros_like(loss_ref)

    loss_ref[...] = loss_ref[...] + part


def kernel(x, enc_w1, enc_b1, enc_w2, enc_b2, codebooks,
           dec_w1, dec_b1, dec_w2, dec_b2, *, interpret=False):
    b = x.shape[0]
    flat = x.reshape(b, IN_DIM)
    block_rows = 512
    grid = (b // block_rows,)

    full = lambda a: pl.BlockSpec(a.shape, lambda i: (0,) * a.ndim)
    recon, idx, loss = pl.pallas_call(
        lambda *refs: _fused_body(*refs, batch_total=float(b)),
        grid=grid,
        in_specs=[
            pl.BlockSpec((block_rows, IN_DIM), lambda i: (i, 0)),
            full(enc_w1),
            pl.BlockSpec((1, 512), lambda i: (0, 0)),
            full(enc_w2),
            pl.BlockSpec((1, 256), lambda i: (0, 0)),
            full(codebooks),
            full(dec_w1),
            pl.BlockSpec((1, 512), lambda i: (0, 0)),
            full(dec_w2),
            pl.BlockSpec((1, IN_DIM), lambda i: (0, 0)),
        ],
        out_specs=[
            pl.BlockSpec((block_rows, IN_DIM), lambda i: (i, 0)),
            pl.BlockSpec((block_rows, NUM_Q), lambda i: (i, 0)),
            pl.BlockSpec((1, 1), lambda i: (0, 0)),
        ],
        out_shape=[
            jax.ShapeDtypeStruct((b, IN_DIM), jnp.float32),
            jax.ShapeDtypeStruct((b, NUM_Q), jnp.int32),
            jax.ShapeDtypeStruct((1, 1), jnp.float32),
        ],
        interpret=interpret,
    )(flat, enc_w1, enc_b1.reshape(1, -1), enc_w2, enc_b2.reshape(1, -1),
      codebooks, dec_w1, dec_b1.reshape(1, -1), dec_w2, dec_b2.reshape(1, -1))

    return (recon.reshape(b, 1, 28, 28), idx, loss[0, 0])
